# Initial kernel scaffold; baseline (speedup 1.0000x reference)
import jax
import jax.numpy as jnp
from jax.experimental import pallas as pl


def kernel(rpn_cls_prob_reshape, rpn_bbox_pred, im_info):
    raise NotImplementedError("write your pallas kernel here")


# speedup vs baseline: 25.9323x; 25.9323x over previous
import jax
import jax.numpy as jnp
from jax import lax
from jax.experimental import pallas as pl
from jax.experimental.pallas import tpu as pltpu

_BBOX_MEAN = (0.000437, 0.002586, -0.123953, -0.081469)
_BBOX_STD = (0.12677, 0.095741, 0.3173, 0.281042)
_ANCHOR_W = (9.232984, 16.0, 27.712813, 18.465969, 32.0, 55.425626,
             36.931937, 64.0, 110.851252, 73.863875, 128.0, 221.702503,
             147.72775, 256.0, 443.405007)
_ANCHOR_H = (27.72668, 16.0, 9.237604, 55.453359, 32.0, 18.475209,
             110.906719, 64.0, 36.950417, 221.813438, 128.0, 73.900834,
             443.626876, 256.0, 147.801669)
_A, _HH, _WW = 15, 34, 34
_N = _A * _HH * _WW
_ROWS = 136
_NP = _ROWS * 128
_THRESH = 0.2
_MINWH = 6.16056
_NMS_T = 0.7
_MAXC = 3000
_C = 3072
_TOPN = 300
_OUTP = 384
_RB = 256
_NBLK = _C // _RB
_CHUNK = 2176
_NCH = _NP // _CHUNK


def _dot(a, b):
    return lax.dot_general(a, b, (((1,), (0,)), ((), ())),
                           preferred_element_type=jnp.float32,
                           precision=lax.Precision.HIGHEST)


def _dot_bf(a, b):
    return lax.dot_general(a, b, (((1,), (0,)), ((), ())),
                           preferred_element_type=jnp.float32)


def _dot_t(a, b):
    return lax.dot_general(a, b, (((1,), (1,)), ((), ())),
                           preferred_element_type=jnp.float32,
                           precision=lax.Precision.HIGHEST)


def _table_lookup(a_i32, table):
    out = jnp.zeros(a_i32.shape, jnp.float32)
    for t, v in enumerate(table):
        out = jnp.where(a_i32 == t, jnp.float32(v), out)
    return out


def _body(meta_ref, data_ref, out_ref, obuf3, acc3, dt3, pm3):
    oh = meta_ref[0]
    ow = meta_ref[1]

    dx = data_ref[0]
    dy = data_ref[1]
    dw = data_ref[2]
    dh = data_ref[3]
    s1 = data_ref[4]
    s0 = data_ref[5]

    k = (lax.broadcasted_iota(jnp.int32, (_ROWS, 128), 0) * 128
         + lax.broadcasted_iota(jnp.int32, (_ROWS, 128), 1))
    a = k % _A
    p = k // _A
    fi = (p // _WW).astype(jnp.float32)
    fj = (p % _WW).astype(jnp.float32)

    aw = _table_lookup(a, _ANCHOR_W)
    ah = _table_lookup(a, _ANCHOR_H)

    sx = 8.0 * fj
    sy = 8.0 * fi
    ax1 = sx - 0.5 * (aw - 1.0)
    ax2 = sx + 0.5 * (aw - 1.0)
    ay1 = sy - 0.5 * (ah - 1.0)
    ay2 = sy + 0.5 * (ah - 1.0)
    widths = ax2 - ax1 + 1.0
    heights = ay2 - ay1 + 1.0
    ctr_x = ax1 + 0.5 * (widths - 1.0)
    ctr_y = ay1 + 0.5 * (heights - 1.0)

    ddx = dx * _BBOX_STD[0] + _BBOX_MEAN[0]
    ddy = dy * _BBOX_STD[1] + _BBOX_MEAN[1]
    ddw = dw * _BBOX_STD[2] + _BBOX_MEAN[2]
    ddh = dh * _BBOX_STD[3] + _BBOX_MEAN[3]

    pcx = ddx * widths + ctr_x
    pcy = ddy * heights + ctr_y
    pw = jnp.exp(ddw) * widths
    ph = jnp.exp(ddh) * heights

    x1 = jnp.clip(pcx - 0.5 * (pw - 1.0), 0.0, ow - 1.0)
    y1 = jnp.clip(pcy - 0.5 * (ph - 1.0), 0.0, oh - 1.0)
    x2 = jnp.clip(pcx + 0.5 * (pw - 1.0), 0.0, ow - 1.0)
    y2 = jnp.clip(pcy + 0.5 * (ph - 1.0), 0.0, oh - 1.0)

    wsz = x2 - x1 + 1.0
    hsz = y2 - y1 + 1.0
    valid = ((s1 > _THRESH) & ((wsz >= _MINWH) | (hsz >= _MINWH))
             & (k < _N))
    score = jnp.where(valid, s1, -jnp.inf)

    nv = jnp.sum(valid.astype(jnp.float32))

    def cnt_gt(v):
        return jnp.sum((score > v).astype(jnp.float32))

    def bs_body(_, c):
        lo, hi = c
        mid = (lo + hi) // 2
        v = lax.bitcast_convert_type(mid, jnp.float32)
        small = cnt_gt(v) < float(_MAXC)
        return (jnp.where(small, lo, mid), jnp.where(small, mid, hi))

    lo0 = jnp.int32(0)
    hi0 = jnp.int32(0x7F000000)
    _, tau_bits = lax.fori_loop(0, 31, bs_body, (lo0, hi0))
    tau = lax.bitcast_convert_type(tau_bits, jnp.float32)
    n_gt = cnt_gt(tau)
    quota = float(_MAXC) - n_gt

    tie = score == tau

    def ts_body(_, c):
        lo, hi = c
        mid = (lo + hi) // 2
        g = jnp.sum((tie & (k < mid)).astype(jnp.float32))
        big = g >= quota
        return (jnp.where(big, lo, mid), jnp.where(big, mid, hi))

    _, m_cut = lax.fori_loop(0, 15, ts_body, (jnp.int32(0), jnp.int32(_NP)))
    sel_topk = (score > tau) | (tie & (k < m_cut))
    big = nv >= float(_MAXC)
    sel = (sel_topk & big) | (valid & jnp.logical_not(big))

    cs = sel.astype(jnp.float32)
    su = (lax.broadcasted_iota(jnp.int32, (128, 128), 0)
          < lax.broadcasted_iota(jnp.int32, (128, 128), 1)).astype(jnp.float32)
    rowpref = _dot(cs, su)
    ll = (lax.broadcasted_iota(jnp.int32, (_ROWS, _ROWS), 1)
          < lax.broadcasted_iota(jnp.int32, (_ROWS, _ROWS), 0)).astype(jnp.float32)
    rowoff = jnp.sum(_dot(ll, cs), axis=1, keepdims=True)
    pos = rowpref + rowoff
    posm = jnp.where(sel, pos, 1e9)

    kf = k.astype(jnp.float32)
    occ = cs
    chans = [x1, y1, x2, y2, s0, s1, kf, occ]
    data_t = jnp.concatenate([c.reshape(1, _NP) for c in chans], axis=0)
    posm_f = posm.reshape(1, _NP)
    for b in range(_NCH):
        dt3[b] = data_t[:, b * _CHUNK:(b + 1) * _CHUNK]
        pm3[b] = posm_f[:, b * _CHUNK:(b + 1) * _CHUNK]

    slot = lax.broadcasted_iota(jnp.int32, (_RB, 1), 0).astype(jnp.float32)

    def scat_out(sb, carry):
        base = (sb * _RB).astype(jnp.float32)

        def scat_in(b, part):
            ohm = (slot + base == pm3[b]).astype(jnp.float32)
            return part + _dot_t(ohm, dt3[b])

        acc3[sb] = lax.fori_loop(0, _NCH, scat_in,
                                 jnp.zeros((_RB, 8), jnp.float32))
        return carry

    lax.fori_loop(0, _NBLK, scat_out, jnp.int32(0))

    acc = acc3[...].reshape(_C, 8)
    acc_t = _dot_t(jnp.eye(8, dtype=jnp.float32), acc)

    x1r = acc_t[0:1, :]
    y1r = acc_t[1:2, :]
    x2r = acc_t[2:3, :]
    y2r = acc_t[3:4, :]
    s1r = acc_t[5:6, :]
    kfr = acc_t[6:7, :]
    ar_r = (x2r - x1r + 1.0) * (y2r - y1r + 1.0)

    def _bet_blk(accb):
        s1b = accb[:, 5:6]
        kfb = accb[:, 6:7]
        return (s1b > s1r) | ((s1b == s1r) & (kfb < kfr))

    def iou_blk(blk, carry):
        accb = acc3[blk]
        x1b = accb[:, 0:1]
        y1b = accb[:, 1:2]
        x2b = accb[:, 2:3]
        y2b = accb[:, 3:4]
        ab = (x2b - x1b + 1.0) * (y2b - y1b + 1.0)
        xx1 = jnp.maximum(x1b, x1r)
        yy1 = jnp.maximum(y1b, y1r)
        xx2 = jnp.minimum(x2b, x2r)
        yy2 = jnp.minimum(y2b, y2r)
        iw = jnp.maximum(0.0, xx2 - xx1 + 1.0)
        ih = jnp.maximum(0.0, yy2 - yy1 + 1.0)
        inter = iw * ih
        ovl = inter > _NMS_T * (ab + ar_r - inter)
        obuf3[blk] = (_bet_blk(accb) & ovl).astype(jnp.bfloat16)
        return carry

    lax.fori_loop(0, _NBLK, iou_blk, jnp.int32(0))

    cval = acc_t[7:8, :]

    def w_cond(c):
        _, it, ch = c
        return ch & (it < _C)

    def w_body(c):
        keep, it, _ = c
        kb = keep.astype(jnp.bfloat16)
        s = _dot_bf(kb[:, 0:_RB], obuf3[0])
        for i in range(1, _NBLK):
            s = s + _dot_bf(kb[:, i * _RB:(i + 1) * _RB], obuf3[i])
        newk = jnp.where(s == 0.0, cval, 0.0)
        ch = jnp.sum(jnp.abs(newk - keep)) > 0.0
        return (newk, it + 1, ch)

    kept, _, _ = lax.while_loop(
        w_cond, w_body, (cval, jnp.int32(0), jnp.bool_(True)))

    kept_bf = kept.astype(jnp.bfloat16)
    rank = _dot_bf(kept_bf[:, 0:_RB], _bet_blk(acc3[0]).astype(jnp.bfloat16))
    for blk in range(1, _NBLK):
        r0 = blk * _RB
        rank = rank + _dot_bf(kept_bf[:, r0:r0 + _RB],
                              _bet_blk(acc3[blk]).astype(jnp.bfloat16))
    riota = lax.broadcasted_iota(jnp.int32, (_OUTP, 1), 0).astype(jnp.float32)
    oh2 = ((rank == riota) & (kept > 0.0)).astype(jnp.float32)
    out_ref[...] = _dot(oh2, acc)


def _rpn_pallas(meta, data, interpret=False):
    return pl.pallas_call(
        _body,
        out_shape=jax.ShapeDtypeStruct((_OUTP, 8), jnp.float32),
        in_specs=[
            pl.BlockSpec(memory_space=pltpu.SMEM),
            pl.BlockSpec(memory_space=pltpu.VMEM),
        ],
        out_specs=pl.BlockSpec(memory_space=pltpu.VMEM),
        scratch_shapes=[
            pltpu.VMEM((_NBLK, _RB, _C), jnp.bfloat16),
            pltpu.VMEM((_NBLK, _RB, 8), jnp.float32),
            pltpu.VMEM((_NCH, 8, _CHUNK), jnp.float32),
            pltpu.VMEM((_NCH, 1, _CHUNK), jnp.float32),
        ],
        interpret=interpret,
    )(meta, data)


def _kernel_impl(rpn_cls_prob_reshape, rpn_bbox_pred, im_info, interpret):
    hw = _HH * _WW
    cls = rpn_cls_prob_reshape.reshape(2, _A, hw).transpose(2, 1, 0)
    bbox = rpn_bbox_pred.reshape(_A, 4, hw).transpose(2, 0, 1)
    pad = _NP - _N

    def prep(x):
        return jnp.pad(x.reshape(-1), (0, pad)).reshape(_ROWS, 128)

    data = jnp.stack([
        prep(bbox[..., 0]), prep(bbox[..., 1]),
        prep(bbox[..., 2]), prep(bbox[..., 3]),
        prep(cls[..., 1]), prep(cls[..., 0]),
        jnp.zeros((_ROWS, 128), jnp.float32),
        jnp.zeros((_ROWS, 128), jnp.float32),
    ])
    meta = im_info.astype(jnp.float32)

    out = _rpn_pallas(meta, data, interpret=interpret)
    fp = out[:_TOPN, 0:4]
    fs = out[:_TOPN, 4:6]
    proposals = jnp.concatenate(
        [jnp.zeros((_TOPN, 1), jnp.float32), fp], axis=1)
    return proposals, fs


def kernel(rpn_cls_prob_reshape, rpn_bbox_pred, im_info):
    return _kernel_impl(rpn_cls_prob_reshape, rpn_bbox_pred, im_info, False)
